# baseline (speedup 1.0000x reference)
import jax
import jax.numpy as jnp
from jax import lax
from jax.experimental import pallas as pl
from jax.experimental.pallas import tpu as pltpu
from jax.experimental.pallas import tpu_sc as plsc

OUT_C = 768
ROW = 384 * 3 * 3
N = OUT_C * ROW
L = 16

NT = 9 * (OUT_C // 8)
TILE_ROW = 8 * 384
NW = 32
T_PER_W = NT // NW
CHUNK_T = 9
N_CHUNKS = T_PER_W // CHUNK_T
CHUNK = CHUNK_T * TILE_ROW
SUBV = 8


def _body(x_hbm, clv_hbm, qps_hbm, al_hbm, wm_hbm, cal_hbm, out_hbm,
          xb0, xb1, cb0, cb1, cal_v, qps_v, al_v, wm_v,
          sx0, sx1, sc0, sc1, so0, so1):
    pltpu.sync_copy(qps_hbm, qps_v)
    pltpu.sync_copy(al_hbm, al_v)
    pltpu.sync_copy(wm_hbm, wm_v)

    alv = al_v[...]
    wmv = wm_v[...]

    qv = qps_v[...]
    lb = [jnp.full((L,), qv[i]) * alv for i in range(8)]
    tb = [(lb[i] + lb[i + 1]) * 0.5 * wmv for i in range(7)]

    wid = lax.axis_index("s") * 2 + lax.axis_index("c")
    t0 = wid * T_PER_W
    pltpu.sync_copy(cal_hbm.at[pl.ds(t0 * 8 * L, T_PER_W * 8 * L)], cal_v)

    xbufs = (xb0, xb1)
    cbufs = (cb0, cb1)
    sxs = (sx0, sx1)
    scs = (sc0, sc1)
    sos = (so0, so1)

    def start_in(k):
        p = k & 1
        base = (t0 + k * CHUNK_T) * TILE_ROW
        dx = pltpu.async_copy(x_hbm.at[pl.ds(base, CHUNK)], xbufs[p], sxs[p])
        dc = pltpu.async_copy(clv_hbm.at[pl.ds(base, CHUNK)], cbufs[p], scs[p])
        return dx, dc

    def compute(k, p):
        xr, cr = xbufs[p], cbufs[p]

        for c in range(3):
            def jbody(jj, c=c):
                vh = jj & 1
                s = (jj >> 1) & 7
                u = jj >> 4
                vbase = u * TILE_ROW + c * 1024 + s * 128 + vh * (4 * L)
                scv = cal_v[pl.ds(k * (CHUNK_T * 8 * L) + u * (8 * L) + s * L,
                                  L)] * wmv
                for v in range(SUBV // 2):
                    off = vbase + v * L
                    w = xr[pl.ds(off, L)] * cr[pl.ds(off, L)]
                    a = jnp.abs(w)
                    q01 = jnp.where(a > tb[0], lb[1], lb[0])
                    q23 = jnp.where(a > tb[2], lb[3], lb[2])
                    q45 = jnp.where(a > tb[4], lb[5], lb[4])
                    q67 = jnp.where(a > tb[6], lb[7], lb[6])
                    qlo = jnp.where(a > tb[1], q23, q01)
                    qhi = jnp.where(a > tb[5], q67, q45)
                    q = jnp.where(a > tb[3], qhi, qlo)
                    cr[pl.ds(off, L)] = q * (jnp.sign(w) * scv)

            plsc.parallel_loop(0, CHUNK_T * 8 * 2, 1, unroll=2)(jbody)

    ins = [start_in(0), start_in(1), None]
    outs = [None, None, None]
    for k in range(N_CHUNKS):
        p = k & 1
        dx, dc = ins[k]
        dx.wait()
        dc.wait()
        compute(k, p)
        base = (t0 + k * CHUNK_T) * TILE_ROW
        outs[k] = pltpu.async_copy(cbufs[p], out_hbm.at[pl.ds(base, CHUNK)],
                                   sos[p])
        if k == 0:
            outs[0].wait()
            ins[2] = start_in(2)
    outs[1].wait()
    outs[2].wait()


@jax.jit
def kernel(x, qps, alpha, wm, clv, calpha):
    def to_linear(a):
        a = jnp.transpose(a, (2, 3, 0, 1)).reshape(NT, 8, 3, 128)
        return jnp.transpose(a, (0, 2, 1, 3)).reshape(N)

    xf = to_linear(x)
    cf = to_linear(clv)
    qps16 = jnp.concatenate([qps, jnp.broadcast_to(qps[7:8], (8,))])
    al16 = jnp.broadcast_to(alpha, (L,))
    wm16 = jnp.broadcast_to(wm, (L,))
    cr = jnp.tile(calpha.reshape(OUT_C // 8, 8), (9, 1))
    cal = jnp.broadcast_to(cr[:, :, None], (NT, 8, L)).reshape(NT * 8 * L)

    mesh = plsc.VectorSubcoreMesh(core_axis_name="c", subcore_axis_name="s")
    f = pl.kernel(
        _body,
        out_type=jax.ShapeDtypeStruct((N,), jnp.float32),
        mesh=mesh,
        scratch_types=[
            pltpu.VMEM((CHUNK,), jnp.float32),
            pltpu.VMEM((CHUNK,), jnp.float32),
            pltpu.VMEM((CHUNK,), jnp.float32),
            pltpu.VMEM((CHUNK,), jnp.float32),
            pltpu.VMEM((T_PER_W * 8 * L,), jnp.float32),
            pltpu.VMEM((L,), jnp.float32),
            pltpu.VMEM((L,), jnp.float32),
            pltpu.VMEM((L,), jnp.float32),
            pltpu.SemaphoreType.DMA,
            pltpu.SemaphoreType.DMA,
            pltpu.SemaphoreType.DMA,
            pltpu.SemaphoreType.DMA,
            pltpu.SemaphoreType.DMA,
            pltpu.SemaphoreType.DMA,
        ],
    )
    out = f(xf, cf, qps16, al16, wm16, cal)

    out = jnp.transpose(out.reshape(NT, 3, 8, 128), (0, 2, 1, 3))
    out = out.reshape(3, 3, OUT_C, 384)
    return jnp.transpose(out, (2, 3, 0, 1))


# speedup vs baseline: 1.3151x; 1.3151x over previous
import jax
import jax.numpy as jnp
from jax import lax
from jax.experimental import pallas as pl
from jax.experimental.pallas import tpu as pltpu
from jax.experimental.pallas import tpu_sc as plsc

OUT_C = 768
ROW = 384 * 3 * 3
N = OUT_C * ROW
L = 16

NT = 9 * (OUT_C // 8)
TILE_ROW = 8 * 384
NW = 32
T_PER_W = NT // NW
CHUNK_T = 9
N_CHUNKS = T_PER_W // CHUNK_T
CHUNK = CHUNK_T * TILE_ROW
SUBV = 8


def _body(x_hbm, clv_hbm, par_hbm, cal_hbm, out_hbm,
          xb0, xb1, cb0, cb1, cal_v, par_v,
          sx0, sx1, sc0, sc1, so0, so1, scal):
    wid = lax.axis_index("s") * 2 + lax.axis_index("c")
    t0 = wid * T_PER_W

    xbufs = (xb0, xb1)
    cbufs = (cb0, cb1)
    sxs = (sx0, sx1)
    scs = (sc0, sc1)
    sos = (so0, so1)

    def start_in(k):
        p = k & 1
        base = (t0 + k * CHUNK_T) * TILE_ROW
        dx = pltpu.async_copy(x_hbm.at[pl.ds(base, CHUNK)], xbufs[p], sxs[p])
        dc = pltpu.async_copy(clv_hbm.at[pl.ds(base, CHUNK)], cbufs[p], scs[p])
        return dx, dc

    in0 = start_in(0)
    in1 = start_in(1)
    dcal = pltpu.async_copy(cal_hbm.at[pl.ds(t0 * 8 * L, T_PER_W * 8 * L)],
                            cal_v, scal)
    pltpu.sync_copy(par_hbm, par_v)

    qv = par_v[pl.ds(0, L)]
    alv = par_v[pl.ds(L, L)]
    wmv = par_v[pl.ds(2 * L, L)]

    lb = [jnp.full((L,), qv[i]) * alv for i in range(8)]
    tb = [(lb[i] + lb[i + 1]) * 0.5 * wmv for i in range(7)]
    dcal.wait()

    def compute(k, p):
        xr, cr = xbufs[p], cbufs[p]

        for c in range(3):
            def jbody(jj, c=c):
                vh = jj & 1
                s = (jj >> 1) & 7
                u = jj >> 4
                vbase = u * TILE_ROW + c * 1024 + s * 128 + vh * (4 * L)
                scv = cal_v[pl.ds(k * (CHUNK_T * 8 * L) + u * (8 * L) + s * L,
                                  L)] * wmv
                for v in range(SUBV // 2):
                    off = vbase + v * L
                    w = xr[pl.ds(off, L)] * cr[pl.ds(off, L)]
                    a = jnp.abs(w)
                    q01 = jnp.where(a > tb[0], lb[1], lb[0])
                    q23 = jnp.where(a > tb[2], lb[3], lb[2])
                    q45 = jnp.where(a > tb[4], lb[5], lb[4])
                    q67 = jnp.where(a > tb[6], lb[7], lb[6])
                    qlo = jnp.where(a > tb[1], q23, q01)
                    qhi = jnp.where(a > tb[5], q67, q45)
                    q = jnp.where(a > tb[3], qhi, qlo)
                    cr[pl.ds(off, L)] = q * (jnp.sign(w) * scv)

            plsc.parallel_loop(0, CHUNK_T * 8 * 2, 1, unroll=1)(jbody)

    ins = [in0, in1, None]
    outs = [None, None, None]
    for k in range(N_CHUNKS):
        p = k & 1
        dx, dc = ins[k]
        dx.wait()
        dc.wait()
        compute(k, p)
        base = (t0 + k * CHUNK_T) * TILE_ROW
        outs[k] = pltpu.async_copy(cbufs[p], out_hbm.at[pl.ds(base, CHUNK)],
                                   sos[p])
        if k == 0:
            outs[0].wait()
            ins[2] = start_in(2)
    outs[1].wait()
    outs[2].wait()


@jax.jit
def kernel(x, qps, alpha, wm, clv, calpha):
    def to_linear(a):
        a = jnp.transpose(a, (2, 3, 0, 1)).reshape(NT, 8, 3, 128)
        return jnp.transpose(a, (0, 2, 1, 3)).reshape(N)

    xf = to_linear(x)
    cf = to_linear(clv)
    par = jnp.concatenate([
        qps, jnp.broadcast_to(qps[7:8], (8,)),
        jnp.broadcast_to(alpha, (L,)),
        jnp.broadcast_to(wm, (L,)),
    ])
    cr = jnp.tile(calpha.reshape(OUT_C // 8, 8), (9, 1))
    cal = jnp.broadcast_to(cr[:, :, None], (NT, 8, L)).reshape(NT * 8 * L)

    mesh = plsc.VectorSubcoreMesh(core_axis_name="c", subcore_axis_name="s")
    f = pl.kernel(
        _body,
        out_type=jax.ShapeDtypeStruct((N,), jnp.float32),
        mesh=mesh,
        scratch_types=[
            pltpu.VMEM((CHUNK,), jnp.float32),
            pltpu.VMEM((CHUNK,), jnp.float32),
            pltpu.VMEM((CHUNK,), jnp.float32),
            pltpu.VMEM((CHUNK,), jnp.float32),
            pltpu.VMEM((T_PER_W * 8 * L,), jnp.float32),
            pltpu.VMEM((3 * L,), jnp.float32),
            pltpu.SemaphoreType.DMA,
            pltpu.SemaphoreType.DMA,
            pltpu.SemaphoreType.DMA,
            pltpu.SemaphoreType.DMA,
            pltpu.SemaphoreType.DMA,
            pltpu.SemaphoreType.DMA,
            pltpu.SemaphoreType.DMA,
        ],
    )
    out = f(xf, cf, par, cal)

    out = jnp.transpose(out.reshape(NT, 3, 8, 128), (0, 2, 1, 3))
    out = out.reshape(3, 3, OUT_C, 384)
    return jnp.transpose(out, (2, 3, 0, 1))
